# baseline (speedup 1.0000x reference)
import functools

import jax
import jax.numpy as jnp
from jax import lax
from jax.experimental import pallas as pl
from jax.experimental.pallas import tpu as pltpu
from jax.experimental.pallas import tpu_sc as plsc

B, S, D = 4, 8192, 768
EPS = 1e-12
L = 16
NW = 32
CH = 16
CHD = CH * D
RB = 8
ROWS_PER_W = S // NW
NCHUNK = ROWS_PER_W // CH
NG = S // CH
NJ = D // L
INV_D = 1.0 / D


def _splat(x, dtype=jnp.float32):
    return jnp.full((L,), x, dtype=dtype)


def _dyn_gather(v, idx):
    return lax.gather(
        v, idx[:, None],
        dimension_numbers=lax.GatherDimensionNumbers(
            offset_dims=(), collapsed_slice_dims=(0,), start_index_map=(0,)),
        slice_sizes=(1,),
        mode=lax.GatherScatterMode.PROMISE_IN_BOUNDS)


def _lane_sum(v):
    iota = lax.iota(jnp.int32, L)
    for k in (1, 2, 4, 8):
        v = v + _dyn_gather(v, jnp.bitwise_xor(iota, k))
    return v


def _newton_rsqrt(v):
    vi = lax.bitcast_convert_type(v, jnp.int32)
    yi = _splat(0x5F3759DF, jnp.int32) - lax.shift_right_logical(vi, _splat(1, jnp.int32))
    y = lax.bitcast_convert_type(yi, jnp.float32)
    half_v = 0.5 * v
    for _ in range(3):
        y = y * (1.5 - half_v * y * y)
    return y


def _sc_body(concat_hbm, type_hbm, pos_hbm, tok_hbm, w_hbm, b_hbm, out_hbm,
             tok_v, tokd_v, w_v, b_v, pos_v, in_v0, in_v1, out_v0, out_v1, t_v,
             sem_in0, sem_in1, sem_out0, sem_out1, sem_pos, sem_t):
    wid = lax.axis_index("s") * 2 + lax.axis_index("c")
    g0 = wid * NCHUNK
    in_refs = (in_v0, in_v1)
    out_refs = (out_v0, out_v1)
    sem_in = (sem_in0, sem_in1)
    sem_out = (sem_out0, sem_out1)

    pltpu.sync_copy(tok_hbm, tok_v)
    pltpu.sync_copy(w_hbm, w_v)
    pltpu.sync_copy(b_hbm, b_v)
    for j in range(NJ):
        dj = pl.ds(j * L, L)
        tokd_v[dj] = tok_v[1, dj] - tok_v[0, dj]

    pltpu.async_copy(pos_hbm.at[g0], pos_v.at[pl.ds(0, CHD)], sem_pos)
    pltpu.async_copy(type_hbm.at[g0], t_v.at[0], sem_t)
    pltpu.async_copy(concat_hbm.at[0, g0], in_v0, sem_in0)
    pltpu.async_copy(concat_hbm.at[1, g0], in_v1, sem_in1)

    def chunk_body(c, _):
        g = g0 + c
        pc = lax.rem(c, 2)
        pcb = pc * CHD
        pltpu.make_async_copy(pos_hbm.at[g], pos_v.at[pl.ds(0, CHD)],
                              sem_pos).wait()
        pltpu.make_async_copy(type_hbm.at[g], t_v.at[pc], sem_t).wait()

        @pl.when(c < NCHUNK - 1)
        def _prefetch_pos():
            pltpu.async_copy(pos_hbm.at[g + 1],
                             pos_v.at[pl.ds((1 - pc) * CHD, CHD)], sem_pos)
            pltpu.async_copy(type_hbm.at[g + 1], t_v.at[1 - pc], sem_t)

        for b in range(B):
            ip = b & 1
            in_ref = in_refs[ip]
            out_ref = out_refs[ip]
            pltpu.make_async_copy(concat_hbm.at[b, g], in_ref,
                                  sem_in[ip]).wait()
            if b >= 2:
                pltpu.make_async_copy(out_ref, out_hbm.at[b, g],
                                      sem_out[ip]).wait()
            else:
                @pl.when(c > 0)
                def _wait_out():
                    pltpu.make_async_copy(out_ref, out_hbm.at[b, g],
                                          sem_out[ip]).wait()
            t_row = t_v[pc, b, :]

            for rb in range(0, CH, RB):
                ts = [_dyn_gather(t_row, _splat(rb + k, jnp.int32))
                      for k in range(RB)]
                zero = _splat(0.0)
                carry0 = (tuple(zero for _ in range(RB)),
                          tuple(zero for _ in range(RB)))

                @plsc.parallel_loop(0, NJ, 1, unroll=2, carry=carry0)
                def pass1(j, carry, ts=ts, rb=rb, in_ref=in_ref):
                    jL = j * L
                    tokd_j = tokd_v[pl.ds(jL, L)]
                    tok0_j = tok_v[0, pl.ds(jL, L)]
                    pos_j = pcb + jL
                    accs, accsqs = carry
                    na, nq = [], []
                    for k in range(RB):
                        rofs = (rb + k) * D
                        x = (in_ref[pl.ds(rofs + jL, L)]
                             + pos_v[pl.ds(pos_j + rofs, L)]
                             + tok0_j + ts[k] * tokd_j)
                        out_ref[pl.ds(rofs + jL, L)] = x
                        na.append(accs[k] + x)
                        nq.append(accsqs[k] + x * x)
                    return tuple(na), tuple(nq)

                accs, accsqs = pass1
                us, ys = [], []
                for k in range(RB):
                    u_spl = _lane_sum(accs[k]) * INV_D
                    var_spl = _lane_sum(accsqs[k]) * INV_D - u_spl * u_spl
                    us.append(u_spl)
                    ys.append(_newton_rsqrt(var_spl + EPS))

                @plsc.parallel_loop(0, NJ, 1, unroll=2)
                def pass2(j, us=us, ys=ys, rb=rb, in_ref=in_ref, out_ref=out_ref):
                    jL = j * L
                    w_j = w_v[pl.ds(jL, L)]
                    b_j = b_v[pl.ds(jL, L)]
                    for k in range(RB):
                        rofs = (rb + k) * D
                        xv = in_ref[pl.ds(rofs + jL, L)]
                        out_ref[pl.ds(rofs + jL, L)] = (
                            (xv - us[k]) * ys[k] * w_j + b_j)

            pltpu.async_copy(out_ref, out_hbm.at[b, g], sem_out[ip])
            if b < 2:
                pltpu.async_copy(concat_hbm.at[b + 2, g], in_ref, sem_in[ip])
            else:
                @pl.when(c < NCHUNK - 1)
                def _prefetch_in():
                    pltpu.async_copy(concat_hbm.at[b - 2, g + 1], in_ref,
                                     sem_in[ip])
        return 0

    lax.fori_loop(0, NCHUNK, chunk_body, 0)
    g_last = g0 + NCHUNK - 1
    pltpu.make_async_copy(out_v0, out_hbm.at[2, g_last], sem_out0).wait()
    pltpu.make_async_copy(out_v1, out_hbm.at[3, g_last], sem_out1).wait()


@jax.jit
def kernel(concat_embeddings, concat_type, pos_emb, tok_emb, ln_weight, ln_bias):
    type_r = (concat_type.astype(jnp.float32)
              .reshape(B, NG, CH).transpose(1, 0, 2))
    pos_r = pos_emb.reshape(NG, CHD)
    concat_r = concat_embeddings.reshape(B, NG, CHD)
    mesh = plsc.VectorSubcoreMesh(core_axis_name="c", subcore_axis_name="s")
    run = functools.partial(
        pl.kernel,
        mesh=mesh,
        out_type=jax.ShapeDtypeStruct((B, NG, CHD), jnp.float32),
        scratch_types=[
            pltpu.VMEM((2, D), jnp.float32),
            pltpu.VMEM((D,), jnp.float32),
            pltpu.VMEM((D,), jnp.float32),
            pltpu.VMEM((D,), jnp.float32),
            pltpu.VMEM((2 * CHD,), jnp.float32),
            pltpu.VMEM((CHD,), jnp.float32),
            pltpu.VMEM((CHD,), jnp.float32),
            pltpu.VMEM((CHD,), jnp.float32),
            pltpu.VMEM((CHD,), jnp.float32),
            pltpu.VMEM((2, B, CH), jnp.float32),
            pltpu.SemaphoreType.DMA,
            pltpu.SemaphoreType.DMA,
            pltpu.SemaphoreType.DMA,
            pltpu.SemaphoreType.DMA,
            pltpu.SemaphoreType.DMA,
            pltpu.SemaphoreType.DMA,
        ],
    )(_sc_body)
    out = run(concat_r, type_r, pos_r, tok_emb, ln_weight, ln_bias)
    return out.reshape(B, S, D)


# speedup vs baseline: 6.4164x; 6.4164x over previous
import jax
import jax.numpy as jnp
from jax.experimental import pallas as pl
from jax.experimental.pallas import tpu as pltpu

B, S, D = 4, 8192, 768
EPS = 1e-12
BS = 512
NS = S // BS


def _body(x_ref, t_ref, pos_ref, tok_ref, w_ref, b_ref, o_ref):
    t = t_ref[0, 0, :]
    tok0 = tok_ref[0, :]
    tokd = tok_ref[1, :] - tok0
    x = x_ref[0] + pos_ref[...] + tok0[None, :] + t[:, None] * tokd[None, :]
    u = jnp.mean(x, axis=-1, keepdims=True)
    xc = x - u
    var = jnp.mean(xc * xc, axis=-1, keepdims=True)
    o_ref[0] = w_ref[...][None, :] * (xc * jax.lax.rsqrt(var + EPS)) + b_ref[...][None, :]


@jax.jit
def kernel(concat_embeddings, concat_type, pos_emb, tok_emb, ln_weight, ln_bias):
    t_f = concat_type.astype(jnp.float32).reshape(B * NS, 1, BS)
    grid = (NS, B)
    out = pl.pallas_call(
        _body,
        grid=grid,
        in_specs=[
            pl.BlockSpec((1, BS, D), lambda s, b: (b, s, 0)),
            pl.BlockSpec((1, 1, BS), lambda s, b: (b * NS + s, 0, 0)),
            pl.BlockSpec((BS, D), lambda s, b: (s, 0)),
            pl.BlockSpec((2, D), lambda s, b: (0, 0)),
            pl.BlockSpec((D,), lambda s, b: (0,)),
            pl.BlockSpec((D,), lambda s, b: (0,)),
        ],
        out_specs=pl.BlockSpec((1, BS, D), lambda s, b: (b, s, 0)),
        out_shape=jax.ShapeDtypeStruct((B, S, D), jnp.float32),
        compiler_params=pltpu.CompilerParams(
            dimension_semantics=("arbitrary", "arbitrary"),
        ),
    )(concat_embeddings, t_f, pos_emb, tok_emb, ln_weight, ln_bias)
    return out


# speedup vs baseline: 7.5586x; 1.1780x over previous
import jax
import jax.numpy as jnp
from jax.experimental import pallas as pl
from jax.experimental.pallas import tpu as pltpu

B, S, D = 4, 8192, 768
EPS = 1e-12
BS = 1024
NS = S // BS


def _body(x_ref, t_ref, pos_ref, tok_ref, w_ref, b_ref, o_ref):
    t = t_ref[0, 0, :]
    tok0 = tok_ref[0, :]
    tokd = tok_ref[1, :] - tok0
    x = x_ref[0] + pos_ref[...] + tok0[None, :] + t[:, None] * tokd[None, :]
    u = jnp.mean(x, axis=-1, keepdims=True)
    xc = x - u
    var = jnp.mean(xc * xc, axis=-1, keepdims=True)
    o_ref[0] = w_ref[...][None, :] * (xc * jax.lax.rsqrt(var + EPS)) + b_ref[...][None, :]


@jax.jit
def kernel(concat_embeddings, concat_type, pos_emb, tok_emb, ln_weight, ln_bias):
    t_f = concat_type.astype(jnp.float32).reshape(B * NS, 1, BS)
    grid = (NS, B)
    out = pl.pallas_call(
        _body,
        grid=grid,
        in_specs=[
            pl.BlockSpec((1, BS, D), lambda s, b: (b, s, 0)),
            pl.BlockSpec((1, 1, BS), lambda s, b: (b * NS + s, 0, 0)),
            pl.BlockSpec((BS, D), lambda s, b: (s, 0)),
            pl.BlockSpec((2, D), lambda s, b: (0, 0)),
            pl.BlockSpec((D,), lambda s, b: (0,)),
            pl.BlockSpec((D,), lambda s, b: (0,)),
        ],
        out_specs=pl.BlockSpec((1, BS, D), lambda s, b: (b, s, 0)),
        out_shape=jax.ShapeDtypeStruct((B, S, D), jnp.float32),
        compiler_params=pltpu.CompilerParams(
            dimension_semantics=("arbitrary", "arbitrary"),
        ),
    )(concat_embeddings, t_f, pos_emb, tok_emb, ln_weight, ln_bias)
    return out


# speedup vs baseline: 8.2806x; 1.0955x over previous
import jax
import jax.numpy as jnp
from jax.experimental import pallas as pl
from jax.experimental.pallas import tpu as pltpu

B, S, D = 4, 8192, 768
EPS = 1e-12
BS = 2048
NS = S // BS


def _body(x_ref, t_ref, pos_ref, tok_ref, w_ref, b_ref, o_ref):
    t = t_ref[0, 0, :]
    tok0 = tok_ref[0, :]
    tokd = tok_ref[1, :] - tok0
    x = x_ref[0] + pos_ref[...] + tok0[None, :] + t[:, None] * tokd[None, :]
    u = jnp.mean(x, axis=-1, keepdims=True)
    xc = x - u
    var = jnp.mean(xc * xc, axis=-1, keepdims=True)
    o_ref[0] = w_ref[...][None, :] * (xc * jax.lax.rsqrt(var + EPS)) + b_ref[...][None, :]


@jax.jit
def kernel(concat_embeddings, concat_type, pos_emb, tok_emb, ln_weight, ln_bias):
    t_f = concat_type.astype(jnp.float32).reshape(B * NS, 1, BS)
    grid = (NS, B)
    out = pl.pallas_call(
        _body,
        grid=grid,
        in_specs=[
            pl.BlockSpec((1, BS, D), lambda s, b: (b, s, 0)),
            pl.BlockSpec((1, 1, BS), lambda s, b: (b * NS + s, 0, 0)),
            pl.BlockSpec((BS, D), lambda s, b: (s, 0)),
            pl.BlockSpec((2, D), lambda s, b: (0, 0)),
            pl.BlockSpec((D,), lambda s, b: (0,)),
            pl.BlockSpec((D,), lambda s, b: (0,)),
        ],
        out_specs=pl.BlockSpec((1, BS, D), lambda s, b: (b, s, 0)),
        out_shape=jax.ShapeDtypeStruct((B, S, D), jnp.float32),
        compiler_params=pltpu.CompilerParams(
            dimension_semantics=("arbitrary", "arbitrary"),
        ),
    )(concat_embeddings, t_f, pos_emb, tok_emb, ln_weight, ln_bias)
    return out
